# Initial kernel scaffold; baseline (speedup 1.0000x reference)
import jax
import jax.numpy as jnp
from jax.experimental import pallas as pl


def kernel(feat, edge_index, globalFeats, isTrain, W1, b1, W2, b2, W3, b3, Wg1, bg1, Wg2, bg2, Wg3, bg3, Wo1, bo1, Wo2, bo2):
    raise NotImplementedError("write your pallas kernel here")


# speedup vs baseline: 11.3250x; 11.3250x over previous
import jax
import jax.numpy as jnp
from jax.experimental import pallas as pl

N_NODES = 54
N_PAD = 64
E_EDGES = 2862
E_PAD = 2864


def _gnn_body(src_ref, dst_ref, feat_ref, g_ref,
              w1_ref, b1_ref, w2_ref, b2_ref, w3_ref, b3_ref,
              wg1_ref, bg1_ref, wg2_ref, bg2_ref, wg3_ref, bg3_ref,
              emb_ref, gout_ref):
    lane = jax.lax.broadcasted_iota(jnp.int32, (E_PAD, N_PAD), 1)
    s_src = (src_ref[...] == lane).astype(jnp.float32)
    s_dst = (dst_ref[...] == lane).astype(jnp.float32)
    a = jax.lax.dot_general(s_dst, s_src, (((0,), (0,)), ((), ())),
                            preferred_element_type=jnp.float32)
    deg_in = jnp.sum(a, axis=1, keepdims=True)
    deg_out = jnp.sum(a, axis=0, keepdims=True)
    rin = jax.lax.rsqrt(jnp.maximum(deg_in, 1.0))
    rout = jax.lax.rsqrt(jnp.maximum(deg_out, 1.0))
    a_norm = a * rin * rout

    h = feat_ref[...]
    x = jnp.dot(h, w1_ref[...], preferred_element_type=jnp.float32)
    h = jnp.maximum(jnp.dot(a_norm, x, preferred_element_type=jnp.float32)
                    + b1_ref[...], 0.0)
    x = jnp.dot(h, w2_ref[...], preferred_element_type=jnp.float32)
    h = jnp.maximum(jnp.dot(a_norm, x, preferred_element_type=jnp.float32)
                    + b2_ref[...], 0.0)
    x = jnp.dot(h, w3_ref[...], preferred_element_type=jnp.float32)
    h = jnp.maximum(jnp.dot(a_norm, x, preferred_element_type=jnp.float32)
                    + b3_ref[...], 0.0)
    emb_ref[...] = h

    g = jnp.maximum(jnp.dot(g_ref[...], wg1_ref[...],
                            preferred_element_type=jnp.float32) + bg1_ref[...], 0.0)
    g = jnp.maximum(jnp.dot(g, wg2_ref[...],
                            preferred_element_type=jnp.float32) + bg2_ref[...], 0.0)
    g = jnp.maximum(jnp.dot(g, wg3_ref[...],
                            preferred_element_type=jnp.float32) + bg3_ref[...], 0.0)
    gout_ref[...] = g


def _head_body(v_ref, wo1_ref, bo1_ref, wo2t_ref, bo2_ref, out_ref):
    h = jnp.maximum(jnp.dot(v_ref[...], wo1_ref[...],
                            preferred_element_type=jnp.float32) + bo1_ref[...], 0.0)
    out = jnp.sum(h * wo2t_ref[...], axis=1, keepdims=True) + bo2_ref[...]
    out_ref[...] = jax.nn.sigmoid(out)


def kernel(feat, edge_index, globalFeats, isTrain, W1, b1, W2, b2, W3, b3,
           Wg1, bg1, Wg2, bg2, Wg3, bg3, Wo1, bo1, Wo2, bo2):
    del isTrain
    ei = edge_index.astype(jnp.int32)
    src = jnp.full((E_PAD, 1), N_PAD - 1, jnp.int32).at[:E_EDGES, 0].set(ei[0])
    dst = jnp.full((E_PAD, 1), N_PAD - 1, jnp.int32).at[:E_EDGES, 0].set(ei[1])
    featp = jnp.zeros((N_PAD, feat.shape[1]), feat.dtype).at[:N_NODES].set(feat)

    emb, g = pl.pallas_call(
        _gnn_body,
        out_shape=(
            jax.ShapeDtypeStruct((N_PAD, W3.shape[1]), jnp.float32),
            jax.ShapeDtypeStruct((1, Wg3.shape[1]), jnp.float32),
        ),
    )(src, dst, featp, globalFeats.reshape(1, -1),
      W1, b1.reshape(1, -1), W2, b2.reshape(1, -1), W3, b3.reshape(1, -1),
      Wg1, bg1.reshape(1, -1), Wg2, bg2.reshape(1, -1), Wg3, bg3.reshape(1, -1))

    v = jnp.concatenate([emb[:N_NODES].reshape(-1), g[0]]).reshape(1, -1)

    out = pl.pallas_call(
        _head_body,
        out_shape=jax.ShapeDtypeStruct((1, 1), jnp.float32),
    )(v, Wo1, bo1.reshape(1, -1), Wo2.reshape(1, -1), bo2.reshape(1, 1))
    return out.reshape(1)


# speedup vs baseline: 17.6340x; 1.5571x over previous
import jax
import jax.numpy as jnp
from jax.experimental import pallas as pl

N_NODES = 54
N_PAD = 64
E_EDGES = 2862
D_EMB = 256
EMB_FLAT = N_NODES * D_EMB


def _net_body(ei_ref, feat_ref, g_ref,
              w1_ref, b1_ref, w2_ref, b2_ref, w3_ref, b3_ref,
              wg1_ref, bg1_ref, wg2_ref, bg2_ref, wg3_ref, bg3_ref,
              wo1_ref, bo1_ref, wo2t_ref, bo2_ref, out_ref):
    ei = ei_ref[...]
    src = jnp.broadcast_to(ei[0], (2 * N_PAD, E_EDGES))
    dst = jnp.broadcast_to(ei[1], (2 * N_PAD, E_EDGES))
    sub = jax.lax.broadcasted_iota(jnp.int32, (2 * N_PAD, E_EDGES), 0)
    st = (jnp.where(sub < N_PAD, src, dst) == (sub & (N_PAD - 1))
          ).astype(jnp.float32)
    m = jax.lax.dot_general(st, st, (((1,), (1,)), ((), ())),
                            preferred_element_type=jnp.float32)
    a = m[N_PAD:, :N_PAD]
    rin = jax.lax.rsqrt(jnp.maximum(jnp.sum(a, axis=1, keepdims=True), 1.0))
    rout = jax.lax.rsqrt(jnp.maximum(jnp.sum(a, axis=0, keepdims=True), 1.0))
    a_norm = a * rin * rout

    x = jnp.dot(feat_ref[...], w1_ref[...], preferred_element_type=jnp.float32)
    h = jnp.maximum(jnp.dot(a_norm[:, :N_NODES], x,
                            preferred_element_type=jnp.float32) + b1_ref[...], 0.0)
    x = jnp.dot(h, w2_ref[...], preferred_element_type=jnp.float32)
    h = jnp.maximum(jnp.dot(a_norm, x,
                            preferred_element_type=jnp.float32) + b2_ref[...], 0.0)
    x = jnp.dot(h, w3_ref[...], preferred_element_type=jnp.float32)
    h = jnp.maximum(jnp.dot(a_norm, x,
                            preferred_element_type=jnp.float32) + b3_ref[...], 0.0)
    row = jax.lax.broadcasted_iota(jnp.int32, (N_PAD, D_EMB), 0)
    emb = jnp.where(row < N_NODES, h, 0.0)

    g = jnp.maximum(jnp.dot(g_ref[...], wg1_ref[...],
                            preferred_element_type=jnp.float32) + bg1_ref[...], 0.0)
    g = jnp.maximum(jnp.dot(g, wg2_ref[...],
                            preferred_element_type=jnp.float32) + bg2_ref[...], 0.0)
    g = jnp.maximum(jnp.dot(g, wg3_ref[...],
                            preferred_element_type=jnp.float32) + bg3_ref[...], 0.0)

    flat = emb.reshape(1, N_PAD * D_EMB)
    o = (jnp.dot(flat[:, :EMB_FLAT], wo1_ref[:EMB_FLAT, :],
                 preferred_element_type=jnp.float32)
         + jnp.dot(g, wo1_ref[EMB_FLAT:, :],
                   preferred_element_type=jnp.float32)
         + bo1_ref[...])
    o = jnp.maximum(o, 0.0)
    o = jnp.sum(o * wo2t_ref[...], axis=1, keepdims=True) + bo2_ref[...]
    out_ref[...] = jax.nn.sigmoid(o)


def kernel(feat, edge_index, globalFeats, isTrain, W1, b1, W2, b2, W3, b3,
           Wg1, bg1, Wg2, bg2, Wg3, bg3, Wo1, bo1, Wo2, bo2):
    del isTrain
    out = pl.pallas_call(
        _net_body,
        out_shape=jax.ShapeDtypeStruct((1, 1), jnp.float32),
    )(edge_index.astype(jnp.int32).reshape(2, 1, E_EDGES),
      feat, globalFeats.reshape(1, -1),
      W1, b1.reshape(1, -1), W2, b2.reshape(1, -1), W3, b3.reshape(1, -1),
      Wg1, bg1.reshape(1, -1), Wg2, bg2.reshape(1, -1), Wg3, bg3.reshape(1, -1),
      Wo1, bo1.reshape(1, -1), Wo2.reshape(1, -1), bo2.reshape(1, 1))
    return out.reshape(1)
